# baseline (speedup 1.0000x reference)
import jax
import jax.numpy as jnp
from jax import lax
from jax.experimental import pallas as pl
from jax.experimental.pallas import tpu as pltpu
from jax.experimental.pallas import tpu_sc as plsc

BATCH = 16384
FEAT = 129
NFEAT = FEAT - 1
EMBED_DIM = 64
OUT_DIM = NFEAT + EMBED_DIM

NC = 2
NS = 16
L = 16
NW = NC * NS

ROWS_PER_W = BATCH // NW
CHUNK = 128
NCHUNK = ROWS_PER_W // CHUNK


def _sc_body(inputs_hbm, flat_hbm, table_hbm, out_hbm,
             pos_v, idxf_v, idx_v, emb_v, sem_f, sem_g):
    wid = lax.axis_index("s") * NC + lax.axis_index("c")
    base = wid * ROWS_PER_W
    feat_d = pltpu.async_copy(
        inputs_hbm.at[pl.ds(base, ROWS_PER_W), pl.ds(0, NFEAT)],
        out_hbm.at[pl.ds(base, ROWS_PER_W), pl.ds(0, NFEAT)], sem_f)
    colbase = base * FEAT + NFEAT
    for j in range(ROWS_PER_W // L):
        pos_v[pl.ds(j * L, L)] = (
            lax.iota(jnp.int32, L) * FEAT + (j * L * FEAT) + colbase)
    id_ds = [pltpu.async_copy(
        flat_hbm.at[pos_v.at[pl.ds(c * CHUNK, CHUNK)]],
        idxf_v.at[pl.ds(c * CHUNK, CHUNK)], sem_g) for c in range(NCHUNK)]
    for d in id_ds:
        d.wait()
    for j in range(ROWS_PER_W // L):
        idx_v[pl.ds(j * L, L)] = idxf_v[pl.ds(j * L, L)].astype(jnp.int32)
    g_ds = [pltpu.async_copy(
        table_hbm.at[idx_v.at[pl.ds(c * CHUNK, CHUNK)]],
        emb_v.at[pl.ds(c * CHUNK, CHUNK), :], sem_g) for c in range(NCHUNK)]
    for d in g_ds:
        d.wait()
    pltpu.sync_copy(
        emb_v, out_hbm.at[pl.ds(base, ROWS_PER_W), pl.ds(NFEAT, EMBED_DIM)])
    feat_d.wait()


@jax.jit
def _personalized_input(inputs, table):
    mesh = plsc.VectorSubcoreMesh(
        core_axis_name="c", subcore_axis_name="s",
        num_cores=NC, num_subcores=NS)
    return pl.kernel(
        _sc_body,
        out_type=jax.ShapeDtypeStruct((BATCH, OUT_DIM), jnp.float32),
        mesh=mesh,
        compiler_params=pltpu.CompilerParams(use_tc_tiling_on_sc=False),
        scratch_types=[
            pltpu.VMEM((ROWS_PER_W,), jnp.int32),
            pltpu.VMEM((ROWS_PER_W,), jnp.float32),
            pltpu.VMEM((ROWS_PER_W,), jnp.int32),
            pltpu.VMEM((ROWS_PER_W, EMBED_DIM), jnp.float32),
            pltpu.SemaphoreType.DMA,
            pltpu.SemaphoreType.DMA,
        ],
    )(inputs, inputs.reshape(-1), table)


def kernel(inputs, table):
    return _personalized_input(inputs, table)


# speedup vs baseline: 3.3898x; 3.3898x over previous
import jax
import jax.numpy as jnp
from jax import lax
from jax.experimental import pallas as pl
from jax.experimental.pallas import tpu as pltpu
from jax.experimental.pallas import tpu_sc as plsc

BATCH = 16384
FEAT = 129
NFEAT = FEAT - 1
EMBED_DIM = 64
OUT_DIM = NFEAT + EMBED_DIM

NC = 2
NS = 16
L = 16
NW = NC * NS

ROWS_PER_W = BATCH // NW
CHUNK = 128
NCHUNK = ROWS_PER_W // CHUNK


def _sc_body(inputs_hbm, flat_hbm, table_hbm, out_hbm,
             pos_v, idxf_v, idx_v, emb_v, feat_v, sem_f, sem_g):
    wid = lax.axis_index("s") * NC + lax.axis_index("c")
    base = wid * ROWS_PER_W
    feat_d = pltpu.async_copy(
        inputs_hbm.at[pl.ds(base, ROWS_PER_W), pl.ds(0, NFEAT)],
        feat_v, sem_f)
    colbase = base * FEAT + NFEAT
    for j in range(ROWS_PER_W // L):
        pos_v[pl.ds(j * L, L)] = (
            lax.iota(jnp.int32, L) * FEAT + (j * L * FEAT) + colbase)
    id_ds = [pltpu.async_copy(
        flat_hbm.at[pos_v.at[pl.ds(c * CHUNK, CHUNK)]],
        idxf_v.at[pl.ds(c * CHUNK, CHUNK)], sem_g) for c in range(NCHUNK)]
    for d in id_ds:
        d.wait()
    for j in range(ROWS_PER_W // L):
        idx_v[pl.ds(j * L, L)] = idxf_v[pl.ds(j * L, L)].astype(jnp.int32)
    g_ds = [pltpu.async_copy(
        table_hbm.at[idx_v.at[pl.ds(c * CHUNK, CHUNK)]],
        emb_v.at[pl.ds(c * CHUNK, CHUNK), :], sem_g)
        for c in range(NCHUNK)]
    feat_d.wait()
    feat_o = pltpu.async_copy(
        feat_v, out_hbm.at[pl.ds(base, ROWS_PER_W), pl.ds(0, NFEAT)], sem_f)
    for d in g_ds:
        d.wait()
    emb_o = pltpu.async_copy(
        emb_v, out_hbm.at[pl.ds(base, ROWS_PER_W), pl.ds(NFEAT, EMBED_DIM)],
        sem_g)
    feat_o.wait()
    emb_o.wait()


@jax.jit
def _personalized_input(inputs, table):
    mesh = plsc.VectorSubcoreMesh(
        core_axis_name="c", subcore_axis_name="s",
        num_cores=NC, num_subcores=NS)
    return pl.kernel(
        _sc_body,
        out_type=jax.ShapeDtypeStruct((BATCH, OUT_DIM), jnp.float32),
        mesh=mesh,
        compiler_params=pltpu.CompilerParams(use_tc_tiling_on_sc=False),
        scratch_types=[
            pltpu.VMEM((ROWS_PER_W,), jnp.int32),
            pltpu.VMEM((ROWS_PER_W,), jnp.float32),
            pltpu.VMEM((ROWS_PER_W,), jnp.int32),
            pltpu.VMEM((ROWS_PER_W, EMBED_DIM), jnp.float32),
            pltpu.VMEM((ROWS_PER_W, NFEAT), jnp.float32),
            pltpu.SemaphoreType.DMA,
            pltpu.SemaphoreType.DMA,
        ],
    )(inputs, inputs.reshape(-1), table)


def kernel(inputs, table):
    return _personalized_input(inputs, table)


# speedup vs baseline: 3.9210x; 1.1567x over previous
import jax
import jax.numpy as jnp
from jax import lax
from jax.experimental import pallas as pl
from jax.experimental.pallas import tpu as pltpu
from jax.experimental.pallas import tpu_sc as plsc

BATCH = 16384
FEAT = 129
NFEAT = FEAT - 1
EMBED_DIM = 64
OUT_DIM = NFEAT + EMBED_DIM

NC = 2
NS = 16
L = 16
NW = NC * NS

ROWS_PER_W = BATCH // NW
CHUNK = 128
NCHUNK = ROWS_PER_W // CHUNK


def _sc_body(inputs_hbm, idx_hbm, table_hbm, out_hbm,
             idx_v, emb_v, feat_v, sem_f, sem_g):
    wid = lax.axis_index("s") * NC + lax.axis_index("c")
    base = wid * ROWS_PER_W
    feat_d = pltpu.async_copy(
        inputs_hbm.at[pl.ds(base, ROWS_PER_W), pl.ds(0, NFEAT)],
        feat_v, sem_f)
    pltpu.sync_copy(idx_hbm.at[pl.ds(base, ROWS_PER_W)], idx_v)
    g_ds = [pltpu.async_copy(
        table_hbm.at[idx_v.at[pl.ds(c * CHUNK, CHUNK)]],
        emb_v.at[pl.ds(c * CHUNK, CHUNK), :], sem_g)
        for c in range(NCHUNK)]
    feat_d.wait()
    feat_o = pltpu.async_copy(
        feat_v, out_hbm.at[pl.ds(base, ROWS_PER_W), pl.ds(0, NFEAT)], sem_f)
    for d in g_ds:
        d.wait()
    emb_o = pltpu.async_copy(
        emb_v, out_hbm.at[pl.ds(base, ROWS_PER_W), pl.ds(NFEAT, EMBED_DIM)],
        sem_g)
    feat_o.wait()
    emb_o.wait()


@jax.jit
def _personalized_input(inputs, table):
    mesh = plsc.VectorSubcoreMesh(
        core_axis_name="c", subcore_axis_name="s",
        num_cores=NC, num_subcores=NS)
    return pl.kernel(
        _sc_body,
        out_type=jax.ShapeDtypeStruct((BATCH, OUT_DIM), jnp.float32),
        mesh=mesh,
        compiler_params=pltpu.CompilerParams(use_tc_tiling_on_sc=False),
        scratch_types=[
            pltpu.VMEM((ROWS_PER_W,), jnp.int32),
            pltpu.VMEM((ROWS_PER_W, EMBED_DIM), jnp.float32),
            pltpu.VMEM((ROWS_PER_W, NFEAT), jnp.float32),
            pltpu.SemaphoreType.DMA,
            pltpu.SemaphoreType.DMA,
        ],
    )(inputs, inputs[:, -1].astype(jnp.int32), table)


def kernel(inputs, table):
    return _personalized_input(inputs, table)


# speedup vs baseline: 5.9290x; 1.5121x over previous
import jax
import jax.numpy as jnp
from jax import lax
from jax.experimental import pallas as pl
from jax.experimental.pallas import tpu as pltpu
from jax.experimental.pallas import tpu_sc as plsc

BATCH = 16384
FEAT = 129
NFEAT = FEAT - 1
EMBED_DIM = 64
OUT_DIM = NFEAT + EMBED_DIM
TPAD = 128

NC = 2
NS = 16
L = 16
NW = NC * NS

ROWS_PER_W = BATCH // NW
CHUNK = 256
NCHUNK = ROWS_PER_W // CHUNK
GCH = 128
NGCH = CHUNK // GCH


def _sc_body(inputs_hbm, idx_hbm, table_hbm, out_hbm,
             idx_v, emb_v, out_v, sem_f, sem_g):
    wid = lax.axis_index("s") * NC + lax.axis_index("c")
    base = wid * ROWS_PER_W
    pltpu.sync_copy(idx_hbm.at[pl.ds(base, ROWS_PER_W)], idx_v)
    for ch in range(NCHUNK):
        rb = base + ch * CHUNK
        feat_d = pltpu.async_copy(
            inputs_hbm.at[pl.ds(rb, CHUNK), pl.ds(0, NFEAT)],
            out_v.at[:, pl.ds(0, NFEAT)], sem_f)
        g_ds = [pltpu.async_copy(
            table_hbm.at[idx_v.at[pl.ds(ch * CHUNK + c * GCH, GCH)]],
            emb_v.at[pl.ds(c * GCH, GCH), :], sem_g)
            for c in range(NGCH)]
        for d in g_ds:
            d.wait()
        feat_d.wait()

        def fold(r, carry):
            for c in range(EMBED_DIM // L):
                out_v[r, pl.ds(NFEAT + c * L, L)] = emb_v[r, pl.ds(c * L, L)]
            return carry

        lax.fori_loop(0, CHUNK, fold, 0)
        pltpu.sync_copy(out_v, out_hbm.at[pl.ds(rb, CHUNK), :])


@jax.jit
def _personalized_input(inputs, table):
    mesh = plsc.VectorSubcoreMesh(
        core_axis_name="c", subcore_axis_name="s",
        num_cores=NC, num_subcores=NS)
    call = pl.kernel(
        _sc_body,
        out_type=jax.ShapeDtypeStruct((BATCH, OUT_DIM), jnp.float32),
        mesh=mesh,
        compiler_params=pltpu.CompilerParams(use_tc_tiling_on_sc=True),
        scratch_types=[
            pltpu.VMEM((ROWS_PER_W,), jnp.int32),
            pltpu.VMEM((CHUNK, TPAD), jnp.float32),
            pltpu.VMEM((CHUNK, OUT_DIM), jnp.float32),
            pltpu.SemaphoreType.DMA,
            pltpu.SemaphoreType.DMA,
        ],
    )
    table_pad = jnp.concatenate(
        [table, jnp.zeros((table.shape[0], TPAD - EMBED_DIM), table.dtype)],
        axis=1)
    return call(inputs, inputs[:, -1].astype(jnp.int32), table_pad)


def kernel(inputs, table):
    return _personalized_input(inputs, table)


# speedup vs baseline: 6.3131x; 1.0648x over previous
import jax
import jax.numpy as jnp
from jax import lax
from jax.experimental import pallas as pl
from jax.experimental.pallas import tpu as pltpu
from jax.experimental.pallas import tpu_sc as plsc

BATCH = 16384
FEAT = 129
NFEAT = FEAT - 1
EMBED_DIM = 64
OUT_DIM = NFEAT + EMBED_DIM
TPAD = 128

NC = 2
NS = 16
L = 16
NW = NC * NS

ROWS_PER_W = BATCH // NW
CHUNK = 128
NCHUNK = ROWS_PER_W // CHUNK
NBUF = 2


def _sc_body(inputs_hbm, idx_hbm, table_hbm, out_hbm,
             idx_v, emb_v, out_v, sem_f, sem_g, sem_o):
    wid = lax.axis_index("s") * NC + lax.axis_index("c")
    base = wid * ROWS_PER_W
    pltpu.sync_copy(idx_hbm.at[pl.ds(base, ROWS_PER_W)], idx_v)

    def issue(ch, b):
        rb = base + ch * CHUNK
        feat_d = pltpu.async_copy(
            inputs_hbm.at[pl.ds(rb, CHUNK), pl.ds(0, NFEAT)],
            out_v.at[b, :, pl.ds(0, NFEAT)], sem_f)
        g_d = pltpu.async_copy(
            table_hbm.at[idx_v.at[pl.ds(ch * CHUNK, CHUNK)]],
            emb_v.at[b], sem_g)
        return feat_d, g_d

    in_ds = [issue(0, 0), issue(1, 1)]
    out_ds = [None, None]
    for ch in range(NCHUNK):
        b = ch % NBUF
        feat_d, g_d = in_ds[b]
        g_d.wait()
        feat_d.wait()

        def fold(r, carry):
            for c in range(EMBED_DIM // L):
                out_v[b, r, pl.ds(NFEAT + c * L, L)] = \
                    emb_v[b, r, pl.ds(c * L, L)]
            return carry

        lax.fori_loop(0, CHUNK, fold, 0)
        out_ds[b] = pltpu.async_copy(
            out_v.at[b], out_hbm.at[pl.ds(base + ch * CHUNK, CHUNK), :],
            sem_o)
        if ch + NBUF < NCHUNK:
            out_ds[b].wait()
            in_ds[b] = issue(ch + NBUF, b)
    for b in range(NBUF):
        if out_ds[b] is not None:
            out_ds[b].wait()


@jax.jit
def _personalized_input(inputs, table):
    mesh = plsc.VectorSubcoreMesh(
        core_axis_name="c", subcore_axis_name="s",
        num_cores=NC, num_subcores=NS)
    call = pl.kernel(
        _sc_body,
        out_type=jax.ShapeDtypeStruct((BATCH, OUT_DIM), jnp.float32),
        mesh=mesh,
        compiler_params=pltpu.CompilerParams(use_tc_tiling_on_sc=True),
        scratch_types=[
            pltpu.VMEM((ROWS_PER_W,), jnp.int32),
            pltpu.VMEM((NBUF, CHUNK, TPAD), jnp.float32),
            pltpu.VMEM((NBUF, CHUNK, OUT_DIM), jnp.float32),
            pltpu.SemaphoreType.DMA,
            pltpu.SemaphoreType.DMA,
            pltpu.SemaphoreType.DMA,
        ],
    )
    table_pad = jnp.concatenate(
        [table, jnp.zeros((table.shape[0], TPAD - EMBED_DIM), table.dtype)],
        axis=1)
    return call(inputs, inputs[:, -1].astype(jnp.int32), table_pad)


def kernel(inputs, table):
    return _personalized_input(inputs, table)
